# baseline (speedup 1.0000x reference)
import jax
import jax.numpy as jnp
from jax import lax
from jax.experimental import pallas as pl
from jax.experimental.pallas import tpu as pltpu

_NUM_SE = 2
_HEADS = 4
_CH = 32
_OUT_DIM = 32
_NT = 4
_HID = 64
_B, _N, _F, _GF = 8, 512, 128, 16
_HC = _HEADS * _CH

_TRN = (((0,), (0,)), ((), ()))


def _fused_body(x_ref, adj_ref, gf_ref, wse_ref, s_ref, d_ref, wfin_ref,
                bfin_ref, w1a_ref, w1b_ref, b1_ref, w2_ref, b2_ref, out_ref):
    x = x_ref[0]
    row_ids = lax.broadcasted_iota(jnp.int32, (_N, _N), 0)
    col_ids = lax.broadcasted_iota(jnp.int32, (_N, _N), 1)
    eyeb = jnp.where(row_ids == col_ids, 1.0, 0.0).astype(jnp.bfloat16)
    eyef = eyeb.astype(jnp.float32)
    ones_col = jnp.ones((_N, 1), jnp.float32)

    fin_pre = jnp.zeros((_N, _OUT_DIM), jnp.float32)
    for se in range(_NUM_SE):
        ab = adj_ref[se, 0].astype(jnp.bfloat16)
        at = lax.dot_general(ab, eyeb, _TRN,
                             preferred_element_type=jnp.float32)
        maskf = jnp.minimum(ab.astype(jnp.float32) + at + eyef, 1.0)
        h = jnp.dot(x, wse_ref[se], preferred_element_type=jnp.float32)
        es = lax.dot_general(h, s_ref[se], (((1,), (0,)), ((), ())),
                             preferred_element_type=jnp.float32)
        ed = lax.dot_general(d_ref[se], h, (((0,), (1,)), ((), ())),
                             preferred_element_type=jnp.float32)
        ed_col = lax.dot_general(h, d_ref[se], (((1,), (0,)), ((), ())),
                                 preferred_element_type=jnp.float32)
        edm_col = jnp.max(ed, axis=1, keepdims=True)
        edm_row = jnp.max(ed_col, axis=0, keepdims=True)
        t = es + edm_row
        s = jnp.maximum(t, 0.0)
        u1 = jnp.exp(t - s)
        u2 = jnp.exp(0.2 * t - s)
        v1 = jnp.exp(ed - edm_col)
        v2 = jnp.exp(0.2 * (ed - edm_col))
        outs = []
        for k in range(_HEADS):
            big = jnp.dot(u1[:, k:k + 1], v1[k:k + 1, :],
                          preferred_element_type=jnp.float32)
            small = jnp.dot(u2[:, k:k + 1], v2[k:k + 1, :],
                            preferred_element_type=jnp.float32)
            p = jnp.maximum(big, small) * maskf
            h_aug = jnp.concatenate(
                [h[:, k * _CH:(k + 1) * _CH], ones_col], axis=1)
            agg = jnp.dot(p, h_aug, preferred_element_type=jnp.float32)
            outs.append(agg[:, :_CH] * (1.0 / agg[:, _CH:_CH + 1]))
        out_se = jnp.maximum(jnp.concatenate(outs, axis=1), 0.0)
        fin_pre = fin_pre + jnp.dot(out_se, wfin_ref[se * _HC:(se + 1) * _HC, :],
                                    preferred_element_type=jnp.float32)

    fin = jnp.maximum(fin_pre + bfin_ref[:], 0.0)
    pooled = jnp.max(fin, axis=0, keepdims=True)
    gf = gf_ref[0]
    h1 = jnp.dot(pooled, w1a_ref[:], preferred_element_type=jnp.float32)
    h1 = h1 + jnp.dot(gf, w1b_ref[:], preferred_element_type=jnp.float32)
    h1 = jnp.maximum(h1 + b1_ref[:], 0.0)
    sc = jnp.dot(h1, w2_ref[:], preferred_element_type=jnp.float32) + b2_ref[:]
    sc = sc - jnp.max(sc, axis=1, keepdims=True)
    ex = jnp.exp(sc)
    out_ref[0] = ex / jnp.sum(ex, axis=1, keepdims=True)


@jax.jit
def kernel(node_features, adjacency, graph_features, W_se, a_src, a_dst,
           W_fin, b_fin, W_dec1, b_dec1, W_dec2, b_dec2):
    head_eye = jnp.eye(_HEADS, dtype=jnp.float32)
    smat = (a_src[:, :, :, None] * head_eye[None, :, None, :]).reshape(
        _NUM_SE, _HC, _HEADS)
    dmat = (a_dst[:, :, :, None] * head_eye[None, :, None, :]).reshape(
        _NUM_SE, _HC, _HEADS)
    bfin_row = b_fin.reshape(1, _OUT_DIM)
    w1cat = jnp.transpose(W_dec1, (1, 0, 2)).reshape(_OUT_DIM + _GF, _NT * _HID)
    w1a = w1cat[:_OUT_DIM]
    w1b = w1cat[_OUT_DIM:]
    b1row = b_dec1.reshape(1, _NT * _HID)
    nt_eye = jnp.eye(_NT, dtype=jnp.float32)
    w2bd = (W_dec2[:, :, 0][:, :, None] * nt_eye[:, None, :]).reshape(
        _NT * _HID, _NT)
    b2row = b_dec2[:, 0].reshape(1, _NT)
    gf3 = graph_features.reshape(_B, 1, _GF)

    full = lambda shape: pl.BlockSpec(shape, lambda b: (0,) * len(shape))
    out = pl.pallas_call(
        _fused_body,
        grid=(_B,),
        in_specs=[
            pl.BlockSpec((1, _N, _F), lambda b: (b, 0, 0)),
            pl.BlockSpec((_NUM_SE, 1, _N, _N), lambda b: (0, b, 0, 0)),
            pl.BlockSpec((1, 1, _GF), lambda b: (b, 0, 0)),
            full((_NUM_SE, _F, _HC)),
            full((_NUM_SE, _HC, _HEADS)),
            full((_NUM_SE, _HC, _HEADS)),
            full((_NUM_SE * _HC, _OUT_DIM)),
            full((1, _OUT_DIM)),
            full((_OUT_DIM, _NT * _HID)),
            full((_GF, _NT * _HID)),
            full((1, _NT * _HID)),
            full((_NT * _HID, _NT)),
            full((1, _NT)),
        ],
        out_specs=pl.BlockSpec((1, 1, _NT), lambda b: (b, 0, 0)),
        out_shape=jax.ShapeDtypeStruct((_B, 1, _NT), jnp.float32),
        compiler_params=pltpu.CompilerParams(
            dimension_semantics=("arbitrary",),
        ),
    )(node_features, adjacency, gf3, W_se, smat, dmat, W_fin, bfin_row,
      w1a, w1b, b1row, w2bd, b2row)
    return out.reshape(_B, _NT)


# speedup vs baseline: 1.1605x; 1.1605x over previous
import jax
import jax.numpy as jnp
from jax import lax
from jax.experimental import pallas as pl
from jax.experimental.pallas import tpu as pltpu

_NUM_SE = 2
_HEADS = 4
_CH = 32
_OUT_DIM = 32
_NT = 4
_HID = 64
_B, _N, _F, _GF = 8, 512, 128, 16
_HC = _HEADS * _CH

_TRN = (((0,), (0,)), ((), ()))


def _fused_body(x_ref, adj_ref, gf_ref, wse_ref, s_ref, d_ref, wfin_ref,
                bfin_ref, w1a_ref, w1b_ref, b1_ref, w2_ref, b2_ref, out_ref):
    x = x_ref[0]
    row_ids = lax.broadcasted_iota(jnp.int32, (_N, _N), 0)
    col_ids = lax.broadcasted_iota(jnp.int32, (_N, _N), 1)
    eyeb = jnp.where(row_ids == col_ids, 1.0, 0.0).astype(jnp.bfloat16)
    eyef = eyeb.astype(jnp.float32)
    ones_col = jnp.ones((_N, 1), jnp.float32)

    ones_row = jnp.ones((1, _N), jnp.float32)
    log2e = jnp.float32(1.4426950408889634)

    fin_pre = jnp.zeros((_N, _OUT_DIM), jnp.float32)
    for se in range(_NUM_SE):
        ab = adj_ref[se, 0].astype(jnp.bfloat16)
        at = lax.dot_general(ab, eyeb, _TRN,
                             preferred_element_type=jnp.float32)
        maskf = jnp.minimum(ab.astype(jnp.float32) + at + eyef, 1.0)
        h = jnp.dot(x, wse_ref[se], preferred_element_type=jnp.float32)
        es = log2e * lax.dot_general(h, s_ref[se], (((1,), (0,)), ((), ())),
                                     preferred_element_type=jnp.float32)
        ed = log2e * lax.dot_general(d_ref[se], h, (((0,), (1,)), ((), ())),
                                     preferred_element_type=jnp.float32)
        ed_max = jnp.max(ed, axis=1, keepdims=True)
        outs = []
        for k in range(_HEADS):
            lhs = jnp.concatenate([es[:, k:k + 1], ones_col], axis=1)
            rhs = jnp.concatenate([ones_row, ed[k:k + 1, :]], axis=0)
            logit = jnp.dot(lhs, rhs, preferred_element_type=jnp.float32)
            logit = jnp.maximum(logit, 0.2 * logit)
            shift = jnp.maximum(es[:, k:k + 1] + ed_max[k:k + 1, :], 0.0)
            p = jnp.exp2(logit - shift) * maskf
            h_aug = jnp.concatenate(
                [h[:, k * _CH:(k + 1) * _CH], ones_col], axis=1)
            agg = jnp.dot(p, h_aug, preferred_element_type=jnp.float32)
            outs.append(agg[:, :_CH] * (1.0 / agg[:, _CH:_CH + 1]))
        out_se = jnp.maximum(jnp.concatenate(outs, axis=1), 0.0)
        fin_pre = fin_pre + jnp.dot(out_se, wfin_ref[se * _HC:(se + 1) * _HC, :],
                                    preferred_element_type=jnp.float32)

    fin = jnp.maximum(fin_pre + bfin_ref[:], 0.0)
    pooled = jnp.max(fin, axis=0, keepdims=True)
    gf = gf_ref[0]
    h1 = jnp.dot(pooled, w1a_ref[:], preferred_element_type=jnp.float32)
    h1 = h1 + jnp.dot(gf, w1b_ref[:], preferred_element_type=jnp.float32)
    h1 = jnp.maximum(h1 + b1_ref[:], 0.0)
    sc = jnp.dot(h1, w2_ref[:], preferred_element_type=jnp.float32) + b2_ref[:]
    sc = sc - jnp.max(sc, axis=1, keepdims=True)
    ex = jnp.exp(sc)
    out_ref[0] = ex / jnp.sum(ex, axis=1, keepdims=True)


@jax.jit
def kernel(node_features, adjacency, graph_features, W_se, a_src, a_dst,
           W_fin, b_fin, W_dec1, b_dec1, W_dec2, b_dec2):
    head_eye = jnp.eye(_HEADS, dtype=jnp.float32)
    smat = (a_src[:, :, :, None] * head_eye[None, :, None, :]).reshape(
        _NUM_SE, _HC, _HEADS)
    dmat = (a_dst[:, :, :, None] * head_eye[None, :, None, :]).reshape(
        _NUM_SE, _HC, _HEADS)
    bfin_row = b_fin.reshape(1, _OUT_DIM)
    w1cat = jnp.transpose(W_dec1, (1, 0, 2)).reshape(_OUT_DIM + _GF, _NT * _HID)
    w1a = w1cat[:_OUT_DIM]
    w1b = w1cat[_OUT_DIM:]
    b1row = b_dec1.reshape(1, _NT * _HID)
    nt_eye = jnp.eye(_NT, dtype=jnp.float32)
    w2bd = (W_dec2[:, :, 0][:, :, None] * nt_eye[:, None, :]).reshape(
        _NT * _HID, _NT)
    b2row = b_dec2[:, 0].reshape(1, _NT)
    gf3 = graph_features.reshape(_B, 1, _GF)

    full = lambda shape: pl.BlockSpec(shape, lambda b: (0,) * len(shape))
    out = pl.pallas_call(
        _fused_body,
        grid=(_B,),
        in_specs=[
            pl.BlockSpec((1, _N, _F), lambda b: (b, 0, 0)),
            pl.BlockSpec((_NUM_SE, 1, _N, _N), lambda b: (0, b, 0, 0)),
            pl.BlockSpec((1, 1, _GF), lambda b: (b, 0, 0)),
            full((_NUM_SE, _F, _HC)),
            full((_NUM_SE, _HC, _HEADS)),
            full((_NUM_SE, _HC, _HEADS)),
            full((_NUM_SE * _HC, _OUT_DIM)),
            full((1, _OUT_DIM)),
            full((_OUT_DIM, _NT * _HID)),
            full((_GF, _NT * _HID)),
            full((1, _NT * _HID)),
            full((_NT * _HID, _NT)),
            full((1, _NT)),
        ],
        out_specs=pl.BlockSpec((1, 1, _NT), lambda b: (b, 0, 0)),
        out_shape=jax.ShapeDtypeStruct((_B, 1, _NT), jnp.float32),
        compiler_params=pltpu.CompilerParams(
            dimension_semantics=("parallel",),
        ),
    )(node_features, adjacency, gf3, W_se, smat, dmat, W_fin, bfin_row,
      w1a, w1b, b1row, w2bd, b2row)
    return out.reshape(_B, _NT)
